# baseline (speedup 1.0000x reference)
import jax
import jax.numpy as jnp
from jax.experimental import pallas as pl
from jax.experimental.pallas import tpu as pltpu

N_R = 4096
N_RP1 = 4096
D_OUT = 32
BM = 256
NB = N_R // BM


def _elu(v):
    return jnp.where(v > 0, v, jnp.exp(jnp.minimum(v, 0.0)) - 1.0)


def _fused_kernel(n11_ref, n21_ref, x1_ref, x2_ref, w1_ref, w2_ref,
                  out_ref, xw1_ref, xw2_ref, up_ref):
    j = pl.program_id(0)
    i = pl.program_id(1)

    @pl.when((j == 0) & (i == 0))
    def _project():
        xw1_ref[...] = jnp.dot(x1_ref[...], w1_ref[...],
                               preferred_element_type=jnp.float32)
        xw2_ref[...] = jnp.dot(x2_ref[...], w2_ref[...],
                               preferred_element_type=jnp.float32)

    @pl.when(j == 0)
    def _sweep_up():
        up = jnp.dot(n11_ref[...], xw1_ref[...],
                     preferred_element_type=jnp.float32)
        up_ref[pl.ds(i * BM, BM), :] = _elu(up)

    @pl.when(j == 1)
    def _sweep_cob():
        cob = jnp.dot(n21_ref[...], xw2_ref[...],
                      preferred_element_type=jnp.float32)
        out_ref[...] = up_ref[pl.ds(i * BM, BM), :] + _elu(cob)


def kernel(x_1, x_2, neighborhood_1_to_1, neighborhood_2_to_1, W1, W2):
    return pl.pallas_call(
        _fused_kernel,
        grid=(2, NB),
        in_specs=[
            pl.BlockSpec((BM, N_R), lambda j, i: (i * (1 - j) + (NB - 1) * j, 0)),
            pl.BlockSpec((BM, N_RP1), lambda j, i: (i * j, 0)),
            pl.BlockSpec((N_R, x_1.shape[1]), lambda j, i: (0, 0)),
            pl.BlockSpec((N_RP1, x_2.shape[1]), lambda j, i: (0, 0)),
            pl.BlockSpec((x_1.shape[1], D_OUT), lambda j, i: (0, 0)),
            pl.BlockSpec((x_2.shape[1], D_OUT), lambda j, i: (0, 0)),
        ],
        out_specs=pl.BlockSpec((BM, D_OUT), lambda j, i: (i, 0)),
        out_shape=jax.ShapeDtypeStruct((N_R, D_OUT), jnp.float32),
        scratch_shapes=[
            pltpu.VMEM((N_R, D_OUT), jnp.float32),
            pltpu.VMEM((N_RP1, D_OUT), jnp.float32),
            pltpu.VMEM((N_R, D_OUT), jnp.float32),
        ],
        compiler_params=pltpu.CompilerParams(
            dimension_semantics=("arbitrary", "arbitrary"),
        ),
    )(neighborhood_1_to_1, neighborhood_2_to_1, x_1, x_2, W1, W2)


# speedup vs baseline: 1.2035x; 1.2035x over previous
import jax
import jax.numpy as jnp
from jax.experimental import pallas as pl
from jax.experimental.pallas import tpu as pltpu

N_R = 4096
N_RP1 = 4096
D_OUT = 32
BM = 256


def _elu(v):
    return jnp.where(v > 0, v, jnp.exp(jnp.minimum(v, 0.0)) - 1.0)


def _fused_kernel(n11_ref, n21_ref, x1_ref, x2_ref, w1_ref, w2_ref,
                  out_ref, xw1_ref, xw2_ref):
    i = pl.program_id(0)

    @pl.when(i == 0)
    def _project():
        xw1_ref[...] = jnp.dot(x1_ref[...], w1_ref[...],
                               preferred_element_type=jnp.float32
                               ).astype(jnp.bfloat16)
        xw2_ref[...] = jnp.dot(x2_ref[...], w2_ref[...],
                               preferred_element_type=jnp.float32
                               ).astype(jnp.bfloat16)

    up = jnp.dot(n11_ref[...].astype(jnp.bfloat16), xw1_ref[...],
                 preferred_element_type=jnp.float32)
    cob = jnp.dot(n21_ref[...].astype(jnp.bfloat16), xw2_ref[...],
                  preferred_element_type=jnp.float32)
    out_ref[...] = _elu(up) + _elu(cob)


def kernel(x_1, x_2, neighborhood_1_to_1, neighborhood_2_to_1, W1, W2):
    grid = (N_R // BM,)
    return pl.pallas_call(
        _fused_kernel,
        grid=grid,
        in_specs=[
            pl.BlockSpec((BM, N_R), lambda i: (i, 0)),
            pl.BlockSpec((BM, N_RP1), lambda i: (i, 0)),
            pl.BlockSpec((N_R, x_1.shape[1]), lambda i: (0, 0)),
            pl.BlockSpec((N_RP1, x_2.shape[1]), lambda i: (0, 0)),
            pl.BlockSpec((x_1.shape[1], D_OUT), lambda i: (0, 0)),
            pl.BlockSpec((x_2.shape[1], D_OUT), lambda i: (0, 0)),
        ],
        out_specs=pl.BlockSpec((BM, D_OUT), lambda i: (i, 0)),
        out_shape=jax.ShapeDtypeStruct((N_R, D_OUT), jnp.float32),
        scratch_shapes=[
            pltpu.VMEM((N_R, D_OUT), jnp.bfloat16),
            pltpu.VMEM((N_RP1, D_OUT), jnp.bfloat16),
        ],
        compiler_params=pltpu.CompilerParams(
            dimension_semantics=("arbitrary",),
        ),
    )(neighborhood_1_to_1, neighborhood_2_to_1, x_1, x_2, W1, W2)
